# baseline (speedup 1.0000x reference)
import functools

import jax
import jax.numpy as jnp
from jax import lax
from jax.experimental import pallas as pl
from jax.experimental.pallas import tpu as pltpu
from jax.experimental.pallas import tpu_sc as plsc

N_NODES = 10000
N_EDGES = 320000
NW = 32
CHUNK = 128
CHUNKS_PER_W = 80
EDGES_PER_W = CHUNKS_PER_W * CHUNK
E_PAD = NW * EDGES_PER_W
NPAD = 10112
ROWS_PER_TILE = NPAD // 16


def _seg_sum_sc(d, with_deg):
    mesh = plsc.VectorSubcoreMesh(core_axis_name="c", subcore_axis_name="s")
    out_type = [jax.ShapeDtypeStruct((2, NPAD, d), jnp.float32)]
    if with_deg:
        out_type.append(jax.ShapeDtypeStruct((NPAD,), jnp.float32))
        out_type.append(jax.ShapeDtypeStruct((NPAD,), jnp.float32))
    scratch = [
        pltpu.VMEM((CHUNKS_PER_W + 2, CHUNK), jnp.int32),
        pltpu.VMEM((CHUNKS_PER_W, CHUNK), jnp.int32),
        pltpu.VMEM((2, CHUNK, d), jnp.float32),
        pltpu.VMEM((ROWS_PER_TILE, d), jnp.float32),
        pltpu.VMEM_SHARED((NPAD, d), jnp.float32),
        pltpu.SemaphoreType.DMA,
        pltpu.SemaphoreType.DMA,
    ]
    if with_deg:
        scratch += [
            pltpu.VMEM((CHUNK,), jnp.float32),
            pltpu.VMEM((ROWS_PER_TILE,), jnp.float32),
            pltpu.VMEM((NPAD,), jnp.float32),
            pltpu.VMEM_SHARED((NPAD,), jnp.float32),
        ]

    def body(y_hbm, src_hbm, dst_hbm, zrow_hbm, *rest):
        if with_deg:
            (zdeg_hbm, acc_out, deg_out0, deg_out1,
             src_all, dst_all, rows_v, stage_v, acc_sh, sem0, sem1,
             ones_v, zdeg_v, deg_rd_v, deg_sh) = rest
        else:
            (acc_out, src_all, dst_all, rows_v, stage_v, acc_sh,
             sem0, sem1) = rest
        sems = (sem0, sem1)
        cid = lax.axis_index("c")
        sid = lax.axis_index("s")
        wid = sid * 2 + cid
        r0 = pl.multiple_of(sid * ROWS_PER_TILE, 8)
        pltpu.sync_copy(src_hbm.at[wid], src_all)
        pltpu.sync_copy(dst_hbm.at[wid], dst_all)
        pltpu.sync_copy(zrow_hbm.at[pl.ds(r0, ROWS_PER_TILE)], stage_v)
        pltpu.sync_copy(stage_v, acc_sh.at[pl.ds(r0, ROWS_PER_TILE)])
        if with_deg:
            pltpu.sync_copy(zdeg_hbm.at[pl.ds(r0, ROWS_PER_TILE)], zdeg_v)
            pltpu.sync_copy(zdeg_v, deg_sh.at[pl.ds(r0, ROWS_PER_TILE)])
            for k in range(CHUNK // 16):
                ones_v[pl.ds(k * 16, 16)] = jnp.ones((16,), jnp.float32)
        plsc.subcore_barrier()

        pltpu.async_copy(y_hbm.at[src_all.at[0]], rows_v.at[0], sem0)
        pltpu.async_copy(y_hbm.at[src_all.at[1]], rows_v.at[1], sem1)

        def step(it, carry):
            i0 = it * 2
            for b in range(2):
                i = i0 + b
                pltpu.make_async_copy(y_hbm.at[src_all.at[i]],
                                      rows_v.at[b], sems[b]).wait()
                pltpu.sync_copy(rows_v.at[b], acc_sh.at[dst_all.at[i]],
                                add=True)
                if with_deg:
                    pltpu.sync_copy(ones_v, deg_sh.at[dst_all.at[i]],
                                    add=True)
                pltpu.async_copy(y_hbm.at[src_all.at[i + 2]],
                                 rows_v.at[b], sems[b])
            return carry

        lax.fori_loop(0, CHUNKS_PER_W // 2, step, 0)
        for b in range(2):
            pltpu.make_async_copy(y_hbm.at[src_all.at[0]],
                                  rows_v.at[b], sems[b]).wait()
        plsc.subcore_barrier()
        pltpu.sync_copy(acc_sh.at[pl.ds(r0, ROWS_PER_TILE)], stage_v)
        pltpu.sync_copy(stage_v, acc_out.at[cid, pl.ds(r0, ROWS_PER_TILE)])
        if with_deg:
            @pl.when(sid == 0)
            def _():
                pltpu.sync_copy(deg_sh, deg_rd_v)

            @pl.when(jnp.logical_and(sid == 0, cid == 0))
            def _():
                pltpu.sync_copy(deg_rd_v, deg_out0)

            @pl.when(jnp.logical_and(sid == 0, cid == 1))
            def _():
                pltpu.sync_copy(deg_rd_v, deg_out1)

    return pl.kernel(body, out_type=out_type, mesh=mesh, scratch_types=scratch,
                     compiler_params=pltpu.CompilerParams(
                         use_tc_tiling_on_sc=False),
                     name=f"seg_sum_sc_d{d}")


def _tc_proj(x_ref, w_ref, y_ref, xr_ref):
    h = jnp.dot(x_ref[...], w_ref[...], preferred_element_type=jnp.float32)
    d = y_ref.shape[1]
    y_ref[...] = h[:, :d]
    xr_ref[...] = h[:, d:]


def _tc_mid(acc_ref, degp_ref, xr_ref, b_ref, w_ref, y1_ref, xr1_ref):
    inv = 1.0 / jnp.maximum(degp_ref[0] + degp_ref[1], 1.0)
    agg = (acc_ref[0] + acc_ref[1]) * inv
    h = jnp.maximum(agg + b_ref[...] + xr_ref[...], 0.0)
    hw = jnp.dot(h, w_ref[...], preferred_element_type=jnp.float32)
    d = y1_ref.shape[1]
    y1_ref[...] = hw[:, :d]
    xr1_ref[...] = hw[:, d:]


def _tc_head(acc_ref, degp_ref, xr_ref, b_ref, wfc_ref, bfc_ref,
             emb_ref, out_ref):
    inv = 1.0 / jnp.maximum(degp_ref[0] + degp_ref[1], 1.0)
    agg = (acc_ref[0] + acc_ref[1]) * inv
    h = jnp.maximum(agg + b_ref[...] + xr_ref[...], 0.0)
    emb_ref[...] = h
    out_ref[...] = jnp.dot(h, wfc_ref[...],
                           preferred_element_type=jnp.float32) + bfc_ref[...]


def kernel(x, edge_index, W_l0, b_l0, W_r0, W_l1, b_l1, W_r1, W_fc, b_fc):
    n = N_NODES
    src = edge_index[0]
    dst = edge_index[1]
    pad = E_PAD - N_EDGES
    srcp = jnp.concatenate([src, jnp.zeros((pad,), jnp.int32)])
    srcp = srcp.reshape(NW, CHUNKS_PER_W, CHUNK)
    srcp = jnp.concatenate(
        [srcp, jnp.zeros((NW, 2, CHUNK), jnp.int32)], axis=1)
    junk = n + (jnp.arange(pad, dtype=jnp.int32) % (NPAD - n))
    dstp = jnp.concatenate([dst, junk]).reshape(NW, CHUNKS_PER_W, CHUNK)
    zrow64 = jnp.zeros((NPAD, 64), jnp.float32)
    zrow32 = jnp.zeros((NPAD, 32), jnp.float32)
    zdeg = jnp.zeros((NPAD,), jnp.float32)

    w0 = jnp.concatenate([W_l0, W_r0], axis=1)
    y0, xr0 = pl.pallas_call(
        _tc_proj,
        out_shape=[jax.ShapeDtypeStruct((n, 64), jnp.float32),
                   jax.ShapeDtypeStruct((n, 64), jnp.float32)],
    )(x, w0)

    accp0, deg0, deg1 = _seg_sum_sc(64, True)(y0, srcp, dstp, zrow64, zdeg)
    degp2 = jnp.stack([deg0, deg1])[:, :n, None]

    w1 = jnp.concatenate([W_l1, W_r1], axis=1)
    y1, xr1 = pl.pallas_call(
        _tc_mid,
        out_shape=[jax.ShapeDtypeStruct((n, 32), jnp.float32),
                   jax.ShapeDtypeStruct((n, 32), jnp.float32)],
    )(accp0[:, :n, :], degp2, xr0, b_l0.reshape(1, 64), w1)

    accp1, = _seg_sum_sc(32, False)(y1, srcp, dstp, zrow32)

    embedding, out = pl.pallas_call(
        _tc_head,
        out_shape=[jax.ShapeDtypeStruct((n, 32), jnp.float32),
                   jax.ShapeDtypeStruct((n, 64), jnp.float32)],
    )(accp1[:, :n, :], degp2, xr1, b_l1.reshape(1, 32), W_fc,
      b_fc.reshape(1, 64))
    return (embedding, out)


# speedup vs baseline: 1.2366x; 1.2366x over previous
import functools

import jax
import jax.numpy as jnp
from jax import lax
from jax.experimental import pallas as pl
from jax.experimental.pallas import tpu as pltpu
from jax.experimental.pallas import tpu_sc as plsc

N_NODES = 10000
N_EDGES = 320000
NW = 32
CHUNK = 128
CHUNKS_PER_W = 80
EDGES_PER_W = CHUNKS_PER_W * CHUNK
E_PAD = NW * EDGES_PER_W
NPAD = 10112
ROWS_PER_TILE = NPAD // 16


def _seg_sum_sc(d, with_deg):
    mesh = plsc.VectorSubcoreMesh(core_axis_name="c", subcore_axis_name="s")
    out_type = [jax.ShapeDtypeStruct((2, NPAD, d), jnp.float32)]
    if with_deg:
        out_type.append(jax.ShapeDtypeStruct((NPAD,), jnp.float32))
        out_type.append(jax.ShapeDtypeStruct((NPAD,), jnp.float32))
    scratch = [
        pltpu.VMEM((CHUNKS_PER_W + 2, CHUNK), jnp.int32),
        pltpu.VMEM((CHUNKS_PER_W, CHUNK), jnp.int32),
        pltpu.VMEM((2, CHUNK, d), jnp.float32),
        pltpu.VMEM((ROWS_PER_TILE, d), jnp.float32),
        pltpu.VMEM_SHARED((NPAD, d), jnp.float32),
        pltpu.SemaphoreType.DMA,
        pltpu.SemaphoreType.DMA,
    ]
    if with_deg:
        scratch += [
            pltpu.VMEM((CHUNK,), jnp.float32),
            pltpu.VMEM((ROWS_PER_TILE,), jnp.float32),
            pltpu.VMEM((NPAD,), jnp.float32),
            pltpu.VMEM_SHARED((NPAD,), jnp.float32),
        ]

    def body(y_hbm, src_hbm, dst_hbm, zrow_hbm, *rest):
        if with_deg:
            (zdeg_hbm, acc_out, deg_out0, deg_out1,
             src_all, dst_all, rows_v, stage_v, acc_sh, sem0, sem1,
             ones_v, zdeg_v, deg_rd_v, deg_sh) = rest
        else:
            (acc_out, src_all, dst_all, rows_v, stage_v, acc_sh,
             sem0, sem1) = rest
        sems = (sem0, sem1)
        cid = lax.axis_index("c")
        sid = lax.axis_index("s")
        wid = sid * 2 + cid
        r0 = pl.multiple_of(sid * ROWS_PER_TILE, 8)
        pltpu.sync_copy(src_hbm.at[wid], src_all)
        pltpu.sync_copy(dst_hbm.at[wid], dst_all)
        pltpu.sync_copy(zrow_hbm.at[pl.ds(r0, ROWS_PER_TILE)], stage_v)
        pltpu.sync_copy(stage_v, acc_sh.at[pl.ds(r0, ROWS_PER_TILE)])
        if with_deg:
            pltpu.sync_copy(zdeg_hbm.at[pl.ds(r0, ROWS_PER_TILE)], zdeg_v)
            pltpu.sync_copy(zdeg_v, deg_sh.at[pl.ds(r0, ROWS_PER_TILE)])
            for k in range(CHUNK // 16):
                ones_v[pl.ds(k * 16, 16)] = jnp.ones((16,), jnp.float32)
        plsc.subcore_barrier()

        def step(i, carry):
            pltpu.async_copy(y_hbm.at[src_all.at[i]], rows_v.at[0],
                             sem0).wait()
            pltpu.sync_copy(rows_v.at[0], acc_sh.at[dst_all.at[i]], add=True)
            if with_deg:
                pltpu.sync_copy(ones_v, deg_sh.at[dst_all.at[i]], add=True)
            return carry

        lax.fori_loop(0, CHUNKS_PER_W, step, 0)
        plsc.subcore_barrier()
        pltpu.sync_copy(acc_sh.at[pl.ds(r0, ROWS_PER_TILE)], stage_v)
        pltpu.sync_copy(stage_v, acc_out.at[cid, pl.ds(r0, ROWS_PER_TILE)])
        if with_deg:
            @pl.when(sid == 0)
            def _():
                pltpu.sync_copy(deg_sh, deg_rd_v)

            @pl.when(jnp.logical_and(sid == 0, cid == 0))
            def _():
                pltpu.sync_copy(deg_rd_v, deg_out0)

            @pl.when(jnp.logical_and(sid == 0, cid == 1))
            def _():
                pltpu.sync_copy(deg_rd_v, deg_out1)

    return pl.kernel(body, out_type=out_type, mesh=mesh, scratch_types=scratch,
                     compiler_params=pltpu.CompilerParams(
                         use_tc_tiling_on_sc=False),
                     name=f"seg_sum_sc_d{d}")


def _tc_proj(x_ref, w_ref, y_ref, xr_ref):
    h = jnp.dot(x_ref[...], w_ref[...], preferred_element_type=jnp.float32)
    d = y_ref.shape[1]
    y_ref[...] = h[:, :d]
    xr_ref[...] = h[:, d:]


def _tc_mid(acc_ref, degp_ref, xr_ref, b_ref, w_ref, y1_ref, xr1_ref):
    inv = 1.0 / jnp.maximum(degp_ref[0] + degp_ref[1], 1.0)
    agg = (acc_ref[0] + acc_ref[1]) * inv
    h = jnp.maximum(agg + b_ref[...] + xr_ref[...], 0.0)
    hw = jnp.dot(h, w_ref[...], preferred_element_type=jnp.float32)
    d = y1_ref.shape[1]
    y1_ref[...] = hw[:, :d]
    xr1_ref[...] = hw[:, d:]


def _tc_head(acc_ref, degp_ref, xr_ref, b_ref, wfc_ref, bfc_ref,
             emb_ref, out_ref):
    inv = 1.0 / jnp.maximum(degp_ref[0] + degp_ref[1], 1.0)
    agg = (acc_ref[0] + acc_ref[1]) * inv
    h = jnp.maximum(agg + b_ref[...] + xr_ref[...], 0.0)
    emb_ref[...] = h
    out_ref[...] = jnp.dot(h, wfc_ref[...],
                           preferred_element_type=jnp.float32) + bfc_ref[...]


def kernel(x, edge_index, W_l0, b_l0, W_r0, W_l1, b_l1, W_r1, W_fc, b_fc):
    n = N_NODES
    src = edge_index[0]
    dst = edge_index[1]
    pad = E_PAD - N_EDGES
    srcp = jnp.concatenate([src, jnp.zeros((pad,), jnp.int32)])
    srcp = srcp.reshape(NW, CHUNKS_PER_W, CHUNK)
    srcp = jnp.concatenate(
        [srcp, jnp.zeros((NW, 2, CHUNK), jnp.int32)], axis=1)
    junk = n + (jnp.arange(pad, dtype=jnp.int32) % (NPAD - n))
    dstp = jnp.concatenate([dst, junk]).reshape(NW, CHUNKS_PER_W, CHUNK)
    zrow64 = jnp.zeros((NPAD, 64), jnp.float32)
    zrow32 = jnp.zeros((NPAD, 32), jnp.float32)
    zdeg = jnp.zeros((NPAD,), jnp.float32)

    w0 = jnp.concatenate([W_l0, W_r0], axis=1)
    y0, xr0 = pl.pallas_call(
        _tc_proj,
        out_shape=[jax.ShapeDtypeStruct((n, 64), jnp.float32),
                   jax.ShapeDtypeStruct((n, 64), jnp.float32)],
    )(x, w0)

    accp0, deg0, deg1 = _seg_sum_sc(64, True)(y0, srcp, dstp, zrow64, zdeg)
    degp2 = jnp.stack([deg0, deg1])[:, :n, None]

    w1 = jnp.concatenate([W_l1, W_r1], axis=1)
    y1, xr1 = pl.pallas_call(
        _tc_mid,
        out_shape=[jax.ShapeDtypeStruct((n, 32), jnp.float32),
                   jax.ShapeDtypeStruct((n, 32), jnp.float32)],
    )(accp0[:, :n, :], degp2, xr0, b_l0.reshape(1, 64), w1)

    accp1, = _seg_sum_sc(32, False)(y1, srcp, dstp, zrow32)

    embedding, out = pl.pallas_call(
        _tc_head,
        out_shape=[jax.ShapeDtypeStruct((n, 32), jnp.float32),
                   jax.ShapeDtypeStruct((n, 64), jnp.float32)],
    )(accp1[:, :n, :], degp2, xr1, b_l1.reshape(1, 32), W_fc,
      b_fc.reshape(1, 64))
    return (embedding, out)
